# baseline (device time: 8115 ns/iter reference)
import jax
import jax.numpy as jnp
from jax import lax
from jax.experimental import pallas as pl
from jax.experimental.pallas import tpu as pltpu

NX, NY, NZ = 2, 4, 4


def kernel(u):
    nx, ny, nz = u.shape

    def body(u_ref, o_ref, sbx, sby, sbz, rbx, rby, rbz, ssem, rsem):
        px = lax.axis_index("x")
        py = lax.axis_index("y")
        pz = lax.axis_index("z")

        sbx[0] = u_ref[0:1, :, :]
        sbx[1] = u_ref[nx - 1 : nx, :, :]
        sby[0] = u_ref[:, 0:1, :]
        sby[1] = u_ref[:, ny - 1 : ny, :]
        sbz[0] = u_ref[:, :, 0:1]
        sbz[1] = u_ref[:, :, nz - 1 : nz]

        sends = [
            (px > 0, (px - 1, py, pz), sbx.at[0], rbx.at[1], 0, 1),
            (px < NX - 1, (px + 1, py, pz), sbx.at[1], rbx.at[0], 1, 0),
            (py > 0, (px, py - 1, pz), sby.at[0], rby.at[1], 2, 3),
            (py < NY - 1, (px, py + 1, pz), sby.at[1], rby.at[0], 3, 2),
            (pz > 0, (px, py, pz - 1), sbz.at[0], rbz.at[1], 4, 5),
            (pz < NZ - 1, (px, py, pz + 1), sbz.at[1], rbz.at[0], 5, 4),
        ]

        barrier = pltpu.get_barrier_semaphore()
        for cond, tgt, _, _, _, _ in sends:
            @pl.when(cond)
            def _(tgt=tgt):
                pl.semaphore_signal(
                    barrier, inc=1,
                    device_id=tgt, device_id_type=pl.DeviceIdType.MESH,
                )
        for cond, _, _, _, _, _ in sends:
            @pl.when(cond)
            def _():
                pl.semaphore_wait(barrier, 1)

        for cond, tgt, src, dst, s_i, r_i in sends:
            @pl.when(cond)
            def _(tgt=tgt, src=src, dst=dst, s_i=s_i, r_i=r_i):
                pltpu.make_async_remote_copy(
                    src_ref=src, dst_ref=dst,
                    send_sem=ssem.at[s_i], recv_sem=rsem.at[r_i],
                    device_id=tgt, device_id_type=pl.DeviceIdType.MESH,
                ).start()

        uu = u_ref[...]
        zx = jnp.zeros((1, ny, nz), jnp.float32)
        zy = jnp.zeros((nx, 1, nz), jnp.float32)
        zz = jnp.zeros((nx, ny, 1), jnp.float32)
        v = (
            jnp.concatenate([zx, uu[: nx - 1]], axis=0)
            + jnp.concatenate([uu[1:], zx], axis=0)
            + jnp.concatenate([zy, uu[:, : ny - 1]], axis=1)
            + jnp.concatenate([uu[:, 1:], zy], axis=1)
            + jnp.concatenate([zz, uu[:, :, : nz - 1]], axis=2)
            + jnp.concatenate([uu[:, :, 1:], zz], axis=2)
            - 6.0 * uu
        )

        o_ref[...] = v

        _all = slice(None)
        recvs = [
            (px > 0, rbx, 0, 0, sbx, (slice(0, 1), _all, _all)),
            (px < NX - 1, rbx, 1, 1, sbx, (slice(nx - 1, nx), _all, _all)),
            (py > 0, rby, 0, 2, sby, (_all, slice(0, 1), _all)),
            (py < NY - 1, rby, 1, 3, sby, (_all, slice(ny - 1, ny), _all)),
            (pz > 0, rbz, 0, 4, sbz, (_all, _all, slice(0, 1))),
            (pz < NZ - 1, rbz, 1, 5, sbz, (_all, _all, slice(nz - 1, nz))),
        ]
        for cond, rb, slot, r_i, sb, idx in recvs:
            @pl.when(cond)
            def _(rb=rb, slot=slot, r_i=r_i, sb=sb, idx=idx):
                pltpu.make_async_remote_copy(
                    src_ref=sb.at[slot], dst_ref=rb.at[slot],
                    send_sem=ssem.at[r_i], recv_sem=rsem.at[r_i],
                    device_id=(px, py, pz),
                    device_id_type=pl.DeviceIdType.MESH,
                ).wait_recv()
                o_ref[idx] = o_ref[idx] + rb[slot]

        bounds = [
            (px == 0, (slice(0, 1), _all, _all), zx),
            (px == NX - 1, (slice(nx - 1, nx), _all, _all), zx),
            (py == 0, (_all, slice(0, 1), _all), zy),
            (py == NY - 1, (_all, slice(ny - 1, ny), _all), zy),
            (pz == 0, (_all, _all, slice(0, 1)), zz),
            (pz == NZ - 1, (_all, _all, slice(nz - 1, nz)), zz),
        ]
        for cond, idx, zeros in bounds:
            @pl.when(cond)
            def _(idx=idx, zeros=zeros):
                o_ref[idx] = zeros

        for cond, tgt, src, dst, s_i, r_i in sends:
            @pl.when(cond)
            def _(tgt=tgt, src=src, dst=dst, s_i=s_i, r_i=r_i):
                pltpu.make_async_remote_copy(
                    src_ref=src, dst_ref=dst,
                    send_sem=ssem.at[s_i], recv_sem=rsem.at[r_i],
                    device_id=tgt, device_id_type=pl.DeviceIdType.MESH,
                ).wait_send()

    return pl.pallas_call(
        body,
        out_shape=jax.ShapeDtypeStruct((nx, ny, nz), jnp.float32),
        in_specs=[pl.BlockSpec(memory_space=pltpu.VMEM)],
        out_specs=pl.BlockSpec(memory_space=pltpu.VMEM),
        scratch_shapes=[
            pltpu.VMEM((2, 1, ny, nz), jnp.float32),
            pltpu.VMEM((2, nx, 1, nz), jnp.float32),
            pltpu.VMEM((2, nx, ny, 1), jnp.float32),
            pltpu.VMEM((2, 1, ny, nz), jnp.float32),
            pltpu.VMEM((2, nx, 1, nz), jnp.float32),
            pltpu.VMEM((2, nx, ny, 1), jnp.float32),
            pltpu.SemaphoreType.DMA((6,)),
            pltpu.SemaphoreType.DMA((6,)),
        ],
        compiler_params=pltpu.CompilerParams(collective_id=0),
    )(u)


# device time: 8006 ns/iter; 1.0136x vs baseline; 1.0136x over previous
import jax
import jax.numpy as jnp
from jax import lax
from jax.experimental import pallas as pl
from jax.experimental.pallas import tpu as pltpu

NX, NY, NZ = 2, 4, 4


def kernel(u):
    nx, ny, nz = u.shape

    def body(u_ref, o_ref, sbx, sby, sbz, rbx, rby, rbz, ssem, rsem):
        px = lax.axis_index("x")
        py = lax.axis_index("y")
        pz = lax.axis_index("z")

        sends = [
            (px > 0, (px - 1, py, pz), sbx.at[0], rbx.at[1], 0, 1),
            (px < NX - 1, (px + 1, py, pz), sbx.at[1], rbx.at[0], 1, 0),
            (py > 0, (px, py - 1, pz), sby.at[0], rby.at[1], 2, 3),
            (py < NY - 1, (px, py + 1, pz), sby.at[1], rby.at[0], 3, 2),
            (pz > 0, (px, py, pz - 1), sbz.at[0], rbz.at[1], 4, 5),
            (pz < NZ - 1, (px, py, pz + 1), sbz.at[1], rbz.at[0], 5, 4),
        ]

        barrier = pltpu.get_barrier_semaphore()
        for cond, tgt, _, _, _, _ in sends:
            @pl.when(cond)
            def _(tgt=tgt):
                pl.semaphore_signal(
                    barrier, inc=1,
                    device_id=tgt, device_id_type=pl.DeviceIdType.MESH,
                )

        sbx[0] = u_ref[0:1, :, :]
        sbx[1] = u_ref[nx - 1 : nx, :, :]
        sby[0] = u_ref[:, 0:1, :]
        sby[1] = u_ref[:, ny - 1 : ny, :]
        sbz[0] = u_ref[:, :, 0:1]
        sbz[1] = u_ref[:, :, nz - 1 : nz]

        for cond, _, _, _, _, _ in sends:
            @pl.when(cond)
            def _():
                pl.semaphore_wait(barrier, 1)

        for cond, tgt, src, dst, s_i, r_i in sends:
            @pl.when(cond)
            def _(tgt=tgt, src=src, dst=dst, s_i=s_i, r_i=r_i):
                pltpu.make_async_remote_copy(
                    src_ref=src, dst_ref=dst,
                    send_sem=ssem.at[s_i], recv_sem=rsem.at[r_i],
                    device_id=tgt, device_id_type=pl.DeviceIdType.MESH,
                ).start()

        uu = u_ref[...]
        zx = jnp.zeros((1, ny, nz), jnp.float32)
        zy = jnp.zeros((nx, 1, nz), jnp.float32)
        zz = jnp.zeros((nx, ny, 1), jnp.float32)
        v = (
            jnp.concatenate([zx, uu[: nx - 1]], axis=0)
            + jnp.concatenate([uu[1:], zx], axis=0)
            + jnp.concatenate([zy, uu[:, : ny - 1]], axis=1)
            + jnp.concatenate([uu[:, 1:], zy], axis=1)
            + jnp.concatenate([zz, uu[:, :, : nz - 1]], axis=2)
            + jnp.concatenate([uu[:, :, 1:], zz], axis=2)
            - 6.0 * uu
        )

        o_ref[...] = v

        _all = slice(None)
        recvs = [
            (px > 0, rbx, 0, 0, sbx, (slice(0, 1), _all, _all)),
            (px < NX - 1, rbx, 1, 1, sbx, (slice(nx - 1, nx), _all, _all)),
            (py > 0, rby, 0, 2, sby, (_all, slice(0, 1), _all)),
            (py < NY - 1, rby, 1, 3, sby, (_all, slice(ny - 1, ny), _all)),
            (pz > 0, rbz, 0, 4, sbz, (_all, _all, slice(0, 1))),
            (pz < NZ - 1, rbz, 1, 5, sbz, (_all, _all, slice(nz - 1, nz))),
        ]
        for cond, rb, slot, r_i, sb, idx in recvs:
            @pl.when(cond)
            def _(rb=rb, slot=slot, r_i=r_i, sb=sb, idx=idx):
                pltpu.make_async_remote_copy(
                    src_ref=sb.at[slot], dst_ref=rb.at[slot],
                    send_sem=ssem.at[r_i], recv_sem=rsem.at[r_i],
                    device_id=(px, py, pz),
                    device_id_type=pl.DeviceIdType.MESH,
                ).wait_recv()
                o_ref[idx] = o_ref[idx] + rb[slot]

        bounds = [
            (px == 0, (slice(0, 1), _all, _all), zx),
            (px == NX - 1, (slice(nx - 1, nx), _all, _all), zx),
            (py == 0, (_all, slice(0, 1), _all), zy),
            (py == NY - 1, (_all, slice(ny - 1, ny), _all), zy),
            (pz == 0, (_all, _all, slice(0, 1)), zz),
            (pz == NZ - 1, (_all, _all, slice(nz - 1, nz)), zz),
        ]
        for cond, idx, zeros in bounds:
            @pl.when(cond)
            def _(idx=idx, zeros=zeros):
                o_ref[idx] = zeros

        for cond, tgt, src, dst, s_i, r_i in sends:
            @pl.when(cond)
            def _(tgt=tgt, src=src, dst=dst, s_i=s_i, r_i=r_i):
                pltpu.make_async_remote_copy(
                    src_ref=src, dst_ref=dst,
                    send_sem=ssem.at[s_i], recv_sem=rsem.at[r_i],
                    device_id=tgt, device_id_type=pl.DeviceIdType.MESH,
                ).wait_send()

    return pl.pallas_call(
        body,
        out_shape=jax.ShapeDtypeStruct((nx, ny, nz), jnp.float32),
        in_specs=[pl.BlockSpec(memory_space=pltpu.VMEM)],
        out_specs=pl.BlockSpec(memory_space=pltpu.VMEM),
        scratch_shapes=[
            pltpu.VMEM((2, 1, ny, nz), jnp.float32),
            pltpu.VMEM((2, nx, 1, nz), jnp.float32),
            pltpu.VMEM((2, nx, ny, 1), jnp.float32),
            pltpu.VMEM((2, 1, ny, nz), jnp.float32),
            pltpu.VMEM((2, nx, 1, nz), jnp.float32),
            pltpu.VMEM((2, nx, ny, 1), jnp.float32),
            pltpu.SemaphoreType.DMA((6,)),
            pltpu.SemaphoreType.DMA((6,)),
        ],
        compiler_params=pltpu.CompilerParams(collective_id=0),
    )(u)
